# baseline (speedup 1.0000x reference)
import functools

import numpy as np
import jax
import jax.numpy as jnp
from jax import lax
from jax.experimental import pallas as pl
from jax.experimental.pallas import tpu as pltpu
from jax.experimental.pallas import tpu_sc as plsc

DIM = 768
N_HEADS = 12
HEAD_DIM = 64
KV_LORA = 256
NUM_EXPERTS = 64
EXPERT_HIDDEN = 256
T = 2048
EPS = 1e-6
TB = 256
MT = 128
NW = 32
TPW = T // NW
SB = 32
GR = 4800
CH = 128
TA = 128


def _rotary_consts():
    freqs = 1.0 / (10000.0 ** (np.arange(0, HEAD_DIM, 2)[: HEAD_DIM // 2]
                               .astype(np.float32) / HEAD_DIM))
    t = np.arange(T, dtype=np.float32)
    f = np.outer(t, freqs)
    cos, sin = np.cos(f), np.sin(f)
    c_rep = np.repeat(cos, 2, axis=1)
    s_rep = np.repeat(sin, 2, axis=1)
    sgn = np.tile(np.array([-1.0, 1.0], np.float32), HEAD_DIM // 2)
    c2 = np.tile(c_rep, (1, N_HEADS))
    s2 = np.tile(s_rep * sgn[None, :], (1, N_HEADS))
    pa = np.zeros((DIM, DIM), np.float32)
    idx = np.arange(DIM)
    swapped = idx ^ 1
    pa[swapped, idx] = 1.0
    m = np.zeros((DIM, N_HEADS), np.float32)
    for h in range(N_HEADS):
        m[h * HEAD_DIM:(h + 1) * HEAD_DIM, h] = 1.0
    return jnp.asarray(c2), jnp.asarray(s2), jnp.asarray(pa), jnp.asarray(m)


def _prologue_kern(x_ref, anw_ref, wq_ref, wkd_ref, wkuk_ref, wkuv_ref,
                   c2_ref, s2_ref, pa_ref, m_ref, qw_ref, kw_ref,
                   q_out, k_out, v_out):
    x = x_ref[...]
    var = jnp.mean(x * x, axis=1, keepdims=True)
    xn = x * lax.rsqrt(var + EPS) * anw_ref[...]
    m = m_ref[...]
    pa = pa_ref[...]
    c2 = c2_ref[...]
    s2 = s2_ref[...]

    def headnorm_rot(z, w_row):
        ssq = jnp.dot(z * z, m, preferred_element_type=jnp.float32) / HEAD_DIM
        rs = lax.rsqrt(ssq + EPS)
        bc = lax.dot_general(rs, m, (((1,), (1,)), ((), ())),
                             preferred_element_type=jnp.float32)
        zn = z * bc * w_row
        return zn * c2 + jnp.dot(zn, pa, preferred_element_type=jnp.float32) * s2

    q0 = jnp.dot(xn, wq_ref[...], preferred_element_type=jnp.float32)
    qr = headnorm_rot(q0, qw_ref[...])
    lat = jnp.dot(xn, wkd_ref[...], preferred_element_type=jnp.float32)
    k0 = jnp.dot(lat, wkuk_ref[...], preferred_element_type=jnp.float32)
    kr = headnorm_rot(k0, kw_ref[...])
    vr = jnp.dot(lat, wkuv_ref[...], preferred_element_type=jnp.float32)
    for h in range(N_HEADS):
        sl = slice(h * HEAD_DIM, (h + 1) * HEAD_DIM)
        q_out[h] = qr[:, sl]
        k_out[h] = kr[:, sl]
        v_out[h] = vr[:, sl]


def _attn_kern(q_ref, k_ref, v_ref, o_ref):
    qt = pl.program_id(1)
    q = (q_ref[0] * (1.0 / np.sqrt(HEAD_DIM).astype(np.float32))
         ).astype(jnp.bfloat16)
    s = lax.dot_general(q, k_ref[0].astype(jnp.bfloat16),
                        (((1,), (1,)), ((), ())),
                        preferred_element_type=jnp.float32)
    row = qt * TA + lax.broadcasted_iota(jnp.int32, (TA, T), 0)
    col = lax.broadcasted_iota(jnp.int32, (TA, T), 1)
    s = jnp.where(col <= row, s, -1e9)
    mx = jnp.max(s, axis=1, keepdims=True)
    p = jnp.exp(s - mx)
    l = jnp.sum(p, axis=1, keepdims=True)
    o = jnp.dot(p.astype(jnp.bfloat16), v_ref[0].astype(jnp.bfloat16),
                preferred_element_type=jnp.float32)
    o_ref[0] = o / l


def _oproj_shared_kern(a_ref, wo_ref, x_ref, fw_ref, w1_ref, w2_ref,
                       hn_out, base_out):
    a = jnp.concatenate([a_ref[i] for i in range(N_HEADS)], axis=1)
    h = x_ref[...] + jnp.dot(a.astype(jnp.bfloat16), wo_ref[...],
                             preferred_element_type=jnp.float32)
    var = jnp.mean(h * h, axis=1, keepdims=True)
    hn = h * lax.rsqrt(var + EPS) * fw_ref[...]
    hn_out[...] = hn
    sh = jnp.dot(hn.astype(jnp.bfloat16), w1_ref[...],
                 preferred_element_type=jnp.float32)
    sh = sh / (1.0 + jnp.exp(-sh))
    base_out[...] = h + jnp.dot(sh.astype(jnp.bfloat16), w2_ref[...],
                                preferred_element_type=jnp.float32)


def _gate_kern(hn_ref, gw_ref, aux_ref, w0_ref, w1_ref,
               pos0_ref, pos1_ref, poff_ref, cnt_ref):
    hn = hn_ref[...]
    logits = jnp.dot(hn, gw_ref[...], preferred_element_type=jnp.float32)
    mx = jnp.max(logits, axis=1, keepdims=True)
    ex = jnp.exp(logits - mx)
    probs = ex / jnp.sum(ex, axis=1, keepdims=True)
    auxv = jnp.sum(jnp.mean(probs, axis=0) * jnp.mean(logits, axis=0)
                   ) * NUM_EXPERTS
    aux_ref[...] = jnp.broadcast_to(auxv, (1, 1))

    iE = lax.broadcasted_iota(jnp.int32, (T, NUM_EXPERTS), 1)
    big = jnp.int32(NUM_EXPERTS)
    i1 = jnp.min(jnp.where(logits == mx, iE, big), axis=1, keepdims=True)
    oh1 = iE == i1
    masked = jnp.where(oh1, -jnp.inf, logits)
    m2 = jnp.max(masked, axis=1, keepdims=True)
    i2 = jnp.min(jnp.where(masked == m2, iE, big), axis=1, keepdims=True)
    oh2 = iE == i2
    p1 = jnp.sum(jnp.where(oh1, probs, 0.0), axis=1, keepdims=True)
    p2 = jnp.sum(jnp.where(oh2, probs, 0.0), axis=1, keepdims=True)
    denom = p1 + p2
    w0_ref[...] = jnp.broadcast_to(p1 / denom, (T, 16))
    w1_ref[...] = jnp.broadcast_to(p2 / denom, (T, 16))

    o1f = oh1.astype(jnp.float32)
    o2f = oh2.astype(jnp.float32)
    cnt = jnp.sum(o1f, axis=0, keepdims=True) + jnp.sum(o2f, axis=0,
                                                        keepdims=True)
    cnt_i = cnt.astype(jnp.int32)
    pcnt_i = ((cnt_i + 7) // 8) * 8
    pcnt = pcnt_i.astype(jnp.float32)
    er = lax.broadcasted_iota(jnp.int32, (NUM_EXPERTS, NUM_EXPERTS), 0)
    ec = lax.broadcasted_iota(jnp.int32, (NUM_EXPERTS, NUM_EXPERTS), 1)
    upper = (er < ec).astype(jnp.float32)
    poff = jnp.dot(pcnt, upper, preferred_element_type=jnp.float32)

    rr = lax.broadcasted_iota(jnp.int32, (CH, CH), 0)
    rc = lax.broadcasted_iota(jnp.int32, (CH, CH), 1)
    lstrict = (rc < rr).astype(jnp.float32)

    def ranks(ohf, carry):
        chunks = []
        for i in range(T // CH):
            blk = ohf[i * CH:(i + 1) * CH]
            chunks.append(jnp.dot(lstrict, blk,
                                  preferred_element_type=jnp.float32) + carry)
            carry = carry + jnp.sum(blk, axis=0, keepdims=True)
        return jnp.concatenate(chunks, axis=0), carry

    rank0, carry = ranks(o1f, jnp.zeros((1, NUM_EXPERTS), jnp.float32))
    rank1, _ = ranks(o2f, carry)
    pos0 = jnp.sum(jnp.where(oh1, poff + rank0, 0.0), axis=1, keepdims=True)
    pos1 = jnp.sum(jnp.where(oh2, poff + rank1, 0.0), axis=1, keepdims=True)
    pos0_ref[...] = pos0.astype(jnp.int32)
    pos1_ref[...] = pos1.astype(jnp.int32)
    poff_ref[...] = poff.astype(jnp.int32)
    cnt_ref[...] = cnt_i


def _expert_kern(poff_ref, cnt_ref, g_ref, w1_ref, w2_ref, y_ref):
    e = pl.program_id(0)
    off = poff_ref[0, e]
    c = cnt_ref[0, e]
    nt = (c + MT - 1) // MT
    w1 = w1_ref[0].astype(jnp.bfloat16)
    w2 = w2_ref[0].astype(jnp.bfloat16)

    def body(j, _):
        base = pl.multiple_of(off + j * MT, 8)
        rows = g_ref[pl.ds(base, MT), :].astype(jnp.bfloat16)
        a = jnp.dot(rows, w1, preferred_element_type=jnp.float32)
        a = a / (1.0 + jnp.exp(-a))
        y_ref[pl.ds(base, MT), :] = jnp.dot(a.astype(jnp.bfloat16), w2,
                                            preferred_element_type=jnp.float32)
        return 0

    lax.fori_loop(0, nt, body, 0)


@functools.cache
def _sc_mesh():
    return plsc.VectorSubcoreMesh(core_axis_name="c", subcore_axis_name="s",
                                  num_cores=2, num_subcores=16)


def _dispatch_sc(hn, pos0d, pos1d):
    k = pl.kernel(
        _dispatch_body,
        out_type=jax.ShapeDtypeStruct((GR, DIM), jnp.float32),
        mesh=_sc_mesh(),
        scratch_types=[
            pltpu.VMEM((TPW,), jnp.int32),
            pltpu.VMEM((TPW, DIM), jnp.float32),
            pltpu.SemaphoreType.DMA,
        ],
    )
    return k(hn, pos0d, pos1d)


def _dispatch_body(hn_hbm, pos0_hbm, pos1_hbm, g_hbm, idx_v, rows_v, sem):
    w = lax.axis_index("s") * 2 + lax.axis_index("c")
    base = w * TPW
    pltpu.sync_copy(hn_hbm.at[pl.ds(base, TPW)], rows_v)
    pltpu.sync_copy(pos0_hbm.at[w], idx_v)
    pltpu.async_copy(rows_v, g_hbm.at[idx_v], sem).wait()
    pltpu.sync_copy(pos1_hbm.at[w], idx_v)
    pltpu.async_copy(rows_v, g_hbm.at[idx_v], sem).wait()


def _combine_sc(base, y, pos0c, pos1c, w0e, w1e):
    k = pl.kernel(
        _combine_body,
        out_type=jax.ShapeDtypeStruct((T, DIM), jnp.float32),
        mesh=_sc_mesh(),
        scratch_types=[
            pltpu.VMEM((SB,), jnp.int32),
            pltpu.VMEM((SB,), jnp.int32),
            pltpu.VMEM((SB, DIM), jnp.float32),
            pltpu.VMEM((SB, DIM), jnp.float32),
            pltpu.VMEM((SB, DIM), jnp.float32),
            pltpu.VMEM((SB, 16), jnp.float32),
            pltpu.VMEM((SB, 16), jnp.float32),
            pltpu.SemaphoreType.DMA,
        ],
    )
    return k(base, y, pos0c, pos1c, w0e, w1e)


def _combine_body(base_hbm, y_hbm, pos0_hbm, pos1_hbm, w0_hbm, w1_hbm, out_hbm,
                  idx0_v, idx1_v, y0_v, y1_v, acc_v, w0_v, w1_v, sem):
    w = lax.axis_index("s") * 2 + lax.axis_index("c")
    for sb in range(TPW // SB):
        tok0 = w * TPW + sb * SB
        ci0 = pltpu.async_copy(pos0_hbm.at[w, sb], idx0_v, sem)
        ci1 = pltpu.async_copy(pos1_hbm.at[w, sb], idx1_v, sem)
        ci0.wait()
        ci1.wait()
        c0 = pltpu.async_copy(y_hbm.at[idx0_v], y0_v, sem)
        c1 = pltpu.async_copy(y_hbm.at[idx1_v], y1_v, sem)
        c2 = pltpu.async_copy(base_hbm.at[pl.ds(tok0, SB)], acc_v, sem)
        c3 = pltpu.async_copy(w0_hbm.at[pl.ds(tok0, SB)], w0_v, sem)
        c4 = pltpu.async_copy(w1_hbm.at[pl.ds(tok0, SB)], w1_v, sem)
        c0.wait()
        c1.wait()
        c2.wait()
        c3.wait()
        c4.wait()

        def row_body(r, _):
            w0s = w0_v[r]
            w1s = w1_v[r]
            for cch in range(DIM // 16):
                sl = pl.ds(cch * 16, 16)
                acc_v[r, sl] = (acc_v[r, sl] + w0s * y0_v[r, sl]
                                + w1s * y1_v[r, sl])
            return 0

        lax.fori_loop(0, SB, row_body, 0)
        pltpu.sync_copy(acc_v, out_hbm.at[pl.ds(tok0, SB)])


def _full(shape):
    return pl.BlockSpec(shape, lambda *_: tuple(0 for _ in shape))


def _row(dim):
    return pl.BlockSpec((1, dim), lambda *_: (0, 0))


def kernel(x, attn_norm_w, wq, w_kv_down, w_kv_up, wo, q_norm_w, k_norm_w,
           ffn_norm_w, gate_w, shared_w1, shared_w2, routed_w1, routed_w2):
    xf = x.reshape(T, DIM)
    c2, s2, pa, m = _rotary_consts()
    wku = w_kv_up.reshape(KV_LORA, 2, N_HEADS * HEAD_DIM)
    wkuk = wku[:, 0]
    wkuv = wku[:, 1]
    qw = jnp.tile(q_norm_w, N_HEADS).reshape(1, DIM)
    kw = jnp.tile(k_norm_w, N_HEADS).reshape(1, DIM)
    anw = attn_norm_w.reshape(1, DIM)
    fw = ffn_norm_w.reshape(1, DIM)

    tile = pl.BlockSpec((TB, DIM), lambda t: (t, 0))
    q, k, v = pl.pallas_call(
        _prologue_kern,
        grid=(T // TB,),
        in_specs=[tile, _row(DIM), _full((DIM, DIM)), _full((DIM, KV_LORA)),
                  _full((KV_LORA, DIM)), _full((KV_LORA, DIM)), tile, tile,
                  _full((DIM, DIM)), _full((DIM, N_HEADS)), _row(DIM),
                  _row(DIM)],
        out_specs=[pl.BlockSpec((N_HEADS, TB, HEAD_DIM),
                                lambda t: (0, t, 0))] * 3,
        out_shape=[jax.ShapeDtypeStruct((N_HEADS, T, HEAD_DIM),
                                        jnp.float32)] * 3,
    )(xf, anw, wq, w_kv_down, wkuk, wkuv, c2, s2, pa, m, qw, kw)

    attn = pl.pallas_call(
        _attn_kern,
        grid=(N_HEADS, T // TA),
        in_specs=[pl.BlockSpec((1, TA, HEAD_DIM), lambda h, t: (h, t, 0)),
                  pl.BlockSpec((1, T, HEAD_DIM), lambda h, t: (h, 0, 0)),
                  pl.BlockSpec((1, T, HEAD_DIM), lambda h, t: (h, 0, 0))],
        out_specs=pl.BlockSpec((1, TA, HEAD_DIM), lambda h, t: (h, t, 0)),
        out_shape=jax.ShapeDtypeStruct((N_HEADS, T, HEAD_DIM), jnp.float32),
    )(q, k, v)

    w1cat = jnp.concatenate([shared_w1[0], shared_w1[1]],
                            axis=1).astype(jnp.bfloat16)
    w2cat = jnp.concatenate([shared_w2[0], shared_w2[1]],
                            axis=0).astype(jnp.bfloat16)
    hn, base = pl.pallas_call(
        _oproj_shared_kern,
        grid=(T // TB,),
        in_specs=[pl.BlockSpec((N_HEADS, TB, HEAD_DIM),
                               lambda t: (0, t, 0)),
                  _full((DIM, DIM)), tile, _row(DIM),
                  _full((DIM, 2 * EXPERT_HIDDEN)),
                  _full((2 * EXPERT_HIDDEN, DIM))],
        out_specs=[tile, tile],
        out_shape=[jax.ShapeDtypeStruct((T, DIM), jnp.float32)] * 2,
    )(attn, wo.astype(jnp.bfloat16), xf, fw, w1cat, w2cat)

    aux, w0e, w1e, pos0, pos1, poff, cnt = pl.pallas_call(
        _gate_kern,
        grid=(1,),
        in_specs=[_full((T, DIM)), _full((DIM, NUM_EXPERTS))],
        out_specs=[_full((1, 1)), _full((T, 16)), _full((T, 16)),
                   _full((T, 1)), _full((T, 1)), _full((1, NUM_EXPERTS)),
                   _full((1, NUM_EXPERTS))],
        out_shape=[
            jax.ShapeDtypeStruct((1, 1), jnp.float32),
            jax.ShapeDtypeStruct((T, 16), jnp.float32),
            jax.ShapeDtypeStruct((T, 16), jnp.float32),
            jax.ShapeDtypeStruct((T, 1), jnp.int32),
            jax.ShapeDtypeStruct((T, 1), jnp.int32),
            jax.ShapeDtypeStruct((1, NUM_EXPERTS), jnp.int32),
            jax.ShapeDtypeStruct((1, NUM_EXPERTS), jnp.int32),
        ],
    )(hn, gate_w)

    pos0d = pos0.reshape(NW, TPW)
    pos1d = pos1.reshape(NW, TPW)
    g = _dispatch_sc(hn, pos0d, pos1d)

    y = pl.pallas_call(
        _expert_kern,
        grid=(NUM_EXPERTS,),
        in_specs=[pl.BlockSpec(memory_space=pltpu.SMEM),
                  pl.BlockSpec(memory_space=pltpu.SMEM),
                  _full((GR, DIM)),
                  pl.BlockSpec((1, DIM, EXPERT_HIDDEN), lambda e: (e, 0, 0)),
                  pl.BlockSpec((1, EXPERT_HIDDEN, DIM), lambda e: (e, 0, 0))],
        out_specs=_full((GR, DIM)),
        out_shape=jax.ShapeDtypeStruct((GR, DIM), jnp.float32),
    )(poff, cnt, g, routed_w1, routed_w2)

    pos0c = pos0.reshape(NW, TPW // SB, SB)
    pos1c = pos1.reshape(NW, TPW // SB, SB)
    out = _combine_sc(base, y, pos0c, pos1c, w0e, w1e)

    return out.reshape(x.shape), aux.reshape(())


# speedup vs baseline: 1.1607x; 1.1607x over previous
import functools

import numpy as np
import jax
import jax.numpy as jnp
from jax import lax
from jax.experimental import pallas as pl
from jax.experimental.pallas import tpu as pltpu
from jax.experimental.pallas import tpu_sc as plsc

DIM = 768
N_HEADS = 12
HEAD_DIM = 64
KV_LORA = 256
NUM_EXPERTS = 64
EXPERT_HIDDEN = 256
T = 2048
EPS = 1e-6
TB = 256
MT = 128
NW = 32
TPW = T // NW
SB = 32
GR = 4800
CH = 128
TA = 256


def _rotary_consts():
    freqs = 1.0 / (10000.0 ** (np.arange(0, HEAD_DIM, 2)[: HEAD_DIM // 2]
                               .astype(np.float32) / HEAD_DIM))
    t = np.arange(T, dtype=np.float32)
    f = np.outer(t, freqs)
    cos, sin = np.cos(f), np.sin(f)
    c_rep = np.repeat(cos, 2, axis=1)
    s_rep = np.repeat(sin, 2, axis=1)
    sgn = np.tile(np.array([-1.0, 1.0], np.float32), HEAD_DIM // 2)
    c2 = np.tile(c_rep, (1, N_HEADS))
    s2 = np.tile(s_rep * sgn[None, :], (1, N_HEADS))
    pa = np.zeros((DIM, DIM), np.float32)
    idx = np.arange(DIM)
    swapped = idx ^ 1
    pa[swapped, idx] = 1.0
    m = np.zeros((DIM, N_HEADS), np.float32)
    for h in range(N_HEADS):
        m[h * HEAD_DIM:(h + 1) * HEAD_DIM, h] = 1.0
    return jnp.asarray(c2), jnp.asarray(s2), jnp.asarray(pa), jnp.asarray(m)


def _prologue_kern(x_ref, anw_ref, wq_ref, wkd_ref, wkuk_ref, wkuv_ref,
                   c2_ref, s2_ref, pa_ref, m_ref, qw_ref, kw_ref,
                   q_out, k_out, v_out):
    x = x_ref[...]
    var = jnp.mean(x * x, axis=1, keepdims=True)
    xn = x * lax.rsqrt(var + EPS) * anw_ref[...]
    m = m_ref[...]
    pa = pa_ref[...]
    c2 = c2_ref[...]
    s2 = s2_ref[...]

    def headnorm_rot(z, w_row):
        ssq = jnp.dot(z * z, m, preferred_element_type=jnp.float32) / HEAD_DIM
        rs = lax.rsqrt(ssq + EPS)
        bc = lax.dot_general(rs, m, (((1,), (1,)), ((), ())),
                             preferred_element_type=jnp.float32)
        zn = z * bc * w_row
        return zn * c2 + jnp.dot(zn, pa, preferred_element_type=jnp.float32) * s2

    q0 = jnp.dot(xn, wq_ref[...], preferred_element_type=jnp.float32)
    qr = headnorm_rot(q0, qw_ref[...])
    lat = jnp.dot(xn, wkd_ref[...], preferred_element_type=jnp.float32)
    k0 = jnp.dot(lat, wkuk_ref[...], preferred_element_type=jnp.float32)
    kr = headnorm_rot(k0, kw_ref[...])
    vr = jnp.dot(lat, wkuv_ref[...], preferred_element_type=jnp.float32)
    for h in range(N_HEADS):
        sl = slice(h * HEAD_DIM, (h + 1) * HEAD_DIM)
        q_out[h] = qr[:, sl]
        k_out[h] = kr[:, sl]
        v_out[h] = vr[:, sl]


def _attn_kern(q_ref, k_ref, v_ref, o_ref):
    qt = pl.program_id(1)
    q = (q_ref[0] * (1.0 / np.sqrt(HEAD_DIM).astype(np.float32))
         ).astype(jnp.bfloat16)
    s = lax.dot_general(q, k_ref[0].astype(jnp.bfloat16),
                        (((1,), (1,)), ((), ())),
                        preferred_element_type=jnp.float32)
    row = qt * TA + lax.broadcasted_iota(jnp.int32, (TA, T), 0)
    col = lax.broadcasted_iota(jnp.int32, (TA, T), 1)
    s = jnp.where(col <= row, s, -1e9)
    mx = jnp.max(s, axis=1, keepdims=True)
    p = jnp.exp(s - mx)
    l = jnp.sum(p, axis=1, keepdims=True)
    o = jnp.dot(p.astype(jnp.bfloat16), v_ref[0].astype(jnp.bfloat16),
                preferred_element_type=jnp.float32)
    o_ref[0] = o / l


def _oproj_shared_kern(a_ref, wo_ref, x_ref, fw_ref, w1_ref, w2_ref,
                       hn_out, base_out):
    a = jnp.concatenate([a_ref[i] for i in range(N_HEADS)], axis=1)
    h = x_ref[...] + jnp.dot(a.astype(jnp.bfloat16), wo_ref[...],
                             preferred_element_type=jnp.float32)
    var = jnp.mean(h * h, axis=1, keepdims=True)
    hn = h * lax.rsqrt(var + EPS) * fw_ref[...]
    hn_out[...] = hn
    sh = jnp.dot(hn.astype(jnp.bfloat16), w1_ref[...],
                 preferred_element_type=jnp.float32)
    sh = sh / (1.0 + jnp.exp(-sh))
    base_out[...] = h + jnp.dot(sh.astype(jnp.bfloat16), w2_ref[...],
                                preferred_element_type=jnp.float32)


def _gate_kern(hn_ref, gw_ref, aux_ref, w0_ref, w1_ref,
               pos0_ref, pos1_ref, poff_ref, cnt_ref):
    hn = hn_ref[...]
    logits = jnp.dot(hn, gw_ref[...], preferred_element_type=jnp.float32)
    mx = jnp.max(logits, axis=1, keepdims=True)
    ex = jnp.exp(logits - mx)
    probs = ex / jnp.sum(ex, axis=1, keepdims=True)
    auxv = jnp.sum(jnp.mean(probs, axis=0) * jnp.mean(logits, axis=0)
                   ) * NUM_EXPERTS
    aux_ref[...] = jnp.broadcast_to(auxv, (1, 1))

    iE = lax.broadcasted_iota(jnp.int32, (T, NUM_EXPERTS), 1)
    big = jnp.int32(NUM_EXPERTS)
    i1 = jnp.min(jnp.where(logits == mx, iE, big), axis=1, keepdims=True)
    oh1 = iE == i1
    masked = jnp.where(oh1, -jnp.inf, logits)
    m2 = jnp.max(masked, axis=1, keepdims=True)
    i2 = jnp.min(jnp.where(masked == m2, iE, big), axis=1, keepdims=True)
    oh2 = iE == i2
    p1 = jnp.sum(jnp.where(oh1, probs, 0.0), axis=1, keepdims=True)
    p2 = jnp.sum(jnp.where(oh2, probs, 0.0), axis=1, keepdims=True)
    denom = p1 + p2
    w0_ref[...] = jnp.broadcast_to(p1 / denom, (T, 16))
    w1_ref[...] = jnp.broadcast_to(p2 / denom, (T, 16))

    o1f = oh1.astype(jnp.float32)
    o2f = oh2.astype(jnp.float32)
    cnt = jnp.sum(o1f, axis=0, keepdims=True) + jnp.sum(o2f, axis=0,
                                                        keepdims=True)
    cnt_i = cnt.astype(jnp.int32)
    pcnt_i = ((cnt_i + 7) // 8) * 8
    pcnt = pcnt_i.astype(jnp.float32)
    er = lax.broadcasted_iota(jnp.int32, (NUM_EXPERTS, NUM_EXPERTS), 0)
    ec = lax.broadcasted_iota(jnp.int32, (NUM_EXPERTS, NUM_EXPERTS), 1)
    upper = (er < ec).astype(jnp.float32)
    poff = jnp.dot(pcnt, upper, preferred_element_type=jnp.float32)

    rr = lax.broadcasted_iota(jnp.int32, (CH, CH), 0)
    rc = lax.broadcasted_iota(jnp.int32, (CH, CH), 1)
    lstrict = (rc < rr).astype(jnp.float32)

    def ranks(ohf, carry):
        chunks = []
        for i in range(T // CH):
            blk = ohf[i * CH:(i + 1) * CH]
            chunks.append(jnp.dot(lstrict, blk,
                                  preferred_element_type=jnp.float32) + carry)
            carry = carry + jnp.sum(blk, axis=0, keepdims=True)
        return jnp.concatenate(chunks, axis=0), carry

    rank0, carry = ranks(o1f, jnp.zeros((1, NUM_EXPERTS), jnp.float32))
    rank1, _ = ranks(o2f, carry)
    pos0 = jnp.sum(jnp.where(oh1, poff + rank0, 0.0), axis=1, keepdims=True)
    pos1 = jnp.sum(jnp.where(oh2, poff + rank1, 0.0), axis=1, keepdims=True)
    pos0_ref[...] = pos0.astype(jnp.int32)
    pos1_ref[...] = pos1.astype(jnp.int32)
    poff_ref[...] = poff.astype(jnp.int32)
    cnt_ref[...] = cnt_i


def _expert_kern(poff_ref, cnt_ref, g_ref, w1_ref, w2_ref, y_ref):
    e = pl.program_id(0)
    off = poff_ref[0, e]
    c = cnt_ref[0, e]
    nt = (c + MT - 1) // MT
    w1 = w1_ref[0].astype(jnp.bfloat16)
    w2 = w2_ref[0].astype(jnp.bfloat16)

    def body(j, _):
        base = pl.multiple_of(off + j * MT, 8)
        rows = g_ref[pl.ds(base, MT), :].astype(jnp.bfloat16)
        a = jnp.dot(rows, w1, preferred_element_type=jnp.float32)
        a = a / (1.0 + jnp.exp(-a))
        y_ref[pl.ds(base, MT), :] = jnp.dot(a.astype(jnp.bfloat16), w2,
                                            preferred_element_type=jnp.float32)
        return 0

    lax.fori_loop(0, nt, body, 0)


@functools.cache
def _sc_mesh():
    return plsc.VectorSubcoreMesh(core_axis_name="c", subcore_axis_name="s",
                                  num_cores=2, num_subcores=16)


def _dispatch_sc(hn, pos0d, pos1d):
    k = pl.kernel(
        _dispatch_body,
        out_type=jax.ShapeDtypeStruct((GR, DIM), jnp.float32),
        mesh=_sc_mesh(),
        scratch_types=[
            pltpu.VMEM((TPW,), jnp.int32),
            pltpu.VMEM((TPW, DIM), jnp.float32),
            pltpu.SemaphoreType.DMA,
        ],
    )
    return k(hn, pos0d, pos1d)


def _dispatch_body(hn_hbm, pos0_hbm, pos1_hbm, g_hbm, idx_v, rows_v, sem):
    w = lax.axis_index("s") * 2 + lax.axis_index("c")
    base = w * TPW
    pltpu.sync_copy(hn_hbm.at[pl.ds(base, TPW)], rows_v)
    pltpu.sync_copy(pos0_hbm.at[w], idx_v)
    pltpu.async_copy(rows_v, g_hbm.at[idx_v], sem).wait()
    pltpu.sync_copy(pos1_hbm.at[w], idx_v)
    pltpu.async_copy(rows_v, g_hbm.at[idx_v], sem).wait()


def _combine_sc(base, y, pos0c, pos1c, w0e, w1e):
    k = pl.kernel(
        _combine_body,
        out_type=jax.ShapeDtypeStruct((T, DIM), jnp.float32),
        mesh=_sc_mesh(),
        scratch_types=[
            pltpu.VMEM((SB,), jnp.int32),
            pltpu.VMEM((SB,), jnp.int32),
            pltpu.VMEM((SB, DIM), jnp.float32),
            pltpu.VMEM((SB, DIM), jnp.float32),
            pltpu.VMEM((SB, DIM), jnp.float32),
            pltpu.VMEM((SB, 16), jnp.float32),
            pltpu.VMEM((SB, 16), jnp.float32),
            pltpu.SemaphoreType.DMA,
        ],
    )
    return k(base, y, pos0c, pos1c, w0e, w1e)


def _combine_body(base_hbm, y_hbm, pos0_hbm, pos1_hbm, w0_hbm, w1_hbm, out_hbm,
                  idx0_v, idx1_v, y0_v, y1_v, acc_v, w0_v, w1_v, sem):
    w = lax.axis_index("s") * 2 + lax.axis_index("c")
    for sb in range(TPW // SB):
        tok0 = w * TPW + sb * SB
        ci0 = pltpu.async_copy(pos0_hbm.at[w, sb], idx0_v, sem)
        ci1 = pltpu.async_copy(pos1_hbm.at[w, sb], idx1_v, sem)
        ci0.wait()
        ci1.wait()
        c0 = pltpu.async_copy(y_hbm.at[idx0_v], y0_v, sem)
        c1 = pltpu.async_copy(y_hbm.at[idx1_v], y1_v, sem)
        c2 = pltpu.async_copy(base_hbm.at[pl.ds(tok0, SB)], acc_v, sem)
        c3 = pltpu.async_copy(w0_hbm.at[pl.ds(tok0, SB)], w0_v, sem)
        c4 = pltpu.async_copy(w1_hbm.at[pl.ds(tok0, SB)], w1_v, sem)
        c0.wait()
        c1.wait()
        c2.wait()
        c3.wait()
        c4.wait()

        def row_body(r, _):
            w0s = w0_v[r]
            w1s = w1_v[r]
            for cch in range(DIM // 16):
                sl = pl.ds(cch * 16, 16)
                acc_v[r, sl] = (acc_v[r, sl] + w0s * y0_v[r, sl]
                                + w1s * y1_v[r, sl])
            return 0

        lax.fori_loop(0, SB, row_body, 0)
        pltpu.sync_copy(acc_v, out_hbm.at[pl.ds(tok0, SB)])


def _full(shape):
    return pl.BlockSpec(shape, lambda *_: tuple(0 for _ in shape))


def _row(dim):
    return pl.BlockSpec((1, dim), lambda *_: (0, 0))


def kernel(x, attn_norm_w, wq, w_kv_down, w_kv_up, wo, q_norm_w, k_norm_w,
           ffn_norm_w, gate_w, shared_w1, shared_w2, routed_w1, routed_w2):
    xf = x.reshape(T, DIM)
    c2, s2, pa, m = _rotary_consts()
    wku = w_kv_up.reshape(KV_LORA, 2, N_HEADS * HEAD_DIM)
    wkuk = wku[:, 0]
    wkuv = wku[:, 1]
    qw = jnp.tile(q_norm_w, N_HEADS).reshape(1, DIM)
    kw = jnp.tile(k_norm_w, N_HEADS).reshape(1, DIM)
    anw = attn_norm_w.reshape(1, DIM)
    fw = ffn_norm_w.reshape(1, DIM)

    tile = pl.BlockSpec((TB, DIM), lambda t: (t, 0))
    q, k, v = pl.pallas_call(
        _prologue_kern,
        grid=(T // TB,),
        in_specs=[tile, _row(DIM), _full((DIM, DIM)), _full((DIM, KV_LORA)),
                  _full((KV_LORA, DIM)), _full((KV_LORA, DIM)), tile, tile,
                  _full((DIM, DIM)), _full((DIM, N_HEADS)), _row(DIM),
                  _row(DIM)],
        out_specs=[pl.BlockSpec((N_HEADS, TB, HEAD_DIM),
                                lambda t: (0, t, 0))] * 3,
        out_shape=[jax.ShapeDtypeStruct((N_HEADS, T, HEAD_DIM),
                                        jnp.float32)] * 3,
    )(xf, anw, wq, w_kv_down, wkuk, wkuv, c2, s2, pa, m, qw, kw)

    attn = pl.pallas_call(
        _attn_kern,
        grid=(N_HEADS, T // TA),
        in_specs=[pl.BlockSpec((1, TA, HEAD_DIM), lambda h, t: (h, t, 0)),
                  pl.BlockSpec((1, T, HEAD_DIM), lambda h, t: (h, 0, 0)),
                  pl.BlockSpec((1, T, HEAD_DIM), lambda h, t: (h, 0, 0))],
        out_specs=pl.BlockSpec((1, TA, HEAD_DIM), lambda h, t: (h, t, 0)),
        out_shape=jax.ShapeDtypeStruct((N_HEADS, T, HEAD_DIM), jnp.float32),
    )(q, k, v)

    w1cat = jnp.concatenate([shared_w1[0], shared_w1[1]],
                            axis=1).astype(jnp.bfloat16)
    w2cat = jnp.concatenate([shared_w2[0], shared_w2[1]],
                            axis=0).astype(jnp.bfloat16)
    hn, base = pl.pallas_call(
        _oproj_shared_kern,
        grid=(T // TB,),
        in_specs=[pl.BlockSpec((N_HEADS, TB, HEAD_DIM),
                               lambda t: (0, t, 0)),
                  _full((DIM, DIM)), tile, _row(DIM),
                  _full((DIM, 2 * EXPERT_HIDDEN)),
                  _full((2 * EXPERT_HIDDEN, DIM))],
        out_specs=[tile, tile],
        out_shape=[jax.ShapeDtypeStruct((T, DIM), jnp.float32)] * 2,
    )(attn, wo.astype(jnp.bfloat16), xf, fw, w1cat, w2cat)

    aux, w0e, w1e, pos0, pos1, poff, cnt = pl.pallas_call(
        _gate_kern,
        grid=(1,),
        in_specs=[_full((T, DIM)), _full((DIM, NUM_EXPERTS))],
        out_specs=[_full((1, 1)), _full((T, 16)), _full((T, 16)),
                   _full((T, 1)), _full((T, 1)), _full((1, NUM_EXPERTS)),
                   _full((1, NUM_EXPERTS))],
        out_shape=[
            jax.ShapeDtypeStruct((1, 1), jnp.float32),
            jax.ShapeDtypeStruct((T, 16), jnp.float32),
            jax.ShapeDtypeStruct((T, 16), jnp.float32),
            jax.ShapeDtypeStruct((T, 1), jnp.int32),
            jax.ShapeDtypeStruct((T, 1), jnp.int32),
            jax.ShapeDtypeStruct((1, NUM_EXPERTS), jnp.int32),
            jax.ShapeDtypeStruct((1, NUM_EXPERTS), jnp.int32),
        ],
    )(hn, gate_w)

    pos0d = pos0.reshape(NW, TPW)
    pos1d = pos1.reshape(NW, TPW)
    g = _dispatch_sc(hn, pos0d, pos1d)

    y = pl.pallas_call(
        _expert_kern,
        grid=(NUM_EXPERTS,),
        in_specs=[pl.BlockSpec(memory_space=pltpu.SMEM),
                  pl.BlockSpec(memory_space=pltpu.SMEM),
                  _full((GR, DIM)),
                  pl.BlockSpec((1, DIM, EXPERT_HIDDEN), lambda e: (e, 0, 0)),
                  pl.BlockSpec((1, EXPERT_HIDDEN, DIM), lambda e: (e, 0, 0))],
        out_specs=_full((GR, DIM)),
        out_shape=jax.ShapeDtypeStruct((GR, DIM), jnp.float32),
    )(poff, cnt, g, routed_w1, routed_w2)

    pos0c = pos0.reshape(NW, TPW // SB, SB)
    pos1c = pos1.reshape(NW, TPW // SB, SB)
    out = _combine_sc(base, y, pos0c, pos1c, w0e, w1e)

    return out.reshape(x.shape), aux.reshape(())


# speedup vs baseline: 1.1662x; 1.0047x over previous
import functools

import numpy as np
import jax
import jax.numpy as jnp
from jax import lax
from jax.experimental import pallas as pl
from jax.experimental.pallas import tpu as pltpu
from jax.experimental.pallas import tpu_sc as plsc

DIM = 768
N_HEADS = 12
HEAD_DIM = 64
KV_LORA = 256
NUM_EXPERTS = 64
EXPERT_HIDDEN = 256
T = 2048
EPS = 1e-6
TB = 256
MT = 128
NW = 32
TPW = T // NW
SB = 32
GR = 4800
CH = 128
TA = 256


def _rotary_consts():
    freqs = 1.0 / (10000.0 ** (np.arange(0, HEAD_DIM, 2)[: HEAD_DIM // 2]
                               .astype(np.float32) / HEAD_DIM))
    t = np.arange(T, dtype=np.float32)
    f = np.outer(t, freqs)
    cos, sin = np.cos(f), np.sin(f)
    c_rep = np.repeat(cos, 2, axis=1)
    s_rep = np.repeat(sin, 2, axis=1)
    sgn = np.tile(np.array([-1.0, 1.0], np.float32), HEAD_DIM // 2)
    c2 = np.tile(c_rep, (1, N_HEADS))
    s2 = np.tile(s_rep * sgn[None, :], (1, N_HEADS))
    pa = np.zeros((DIM, DIM), np.float32)
    idx = np.arange(DIM)
    swapped = idx ^ 1
    pa[swapped, idx] = 1.0
    m = np.zeros((DIM, N_HEADS), np.float32)
    for h in range(N_HEADS):
        m[h * HEAD_DIM:(h + 1) * HEAD_DIM, h] = 1.0
    return (jnp.asarray(c2, jnp.bfloat16), jnp.asarray(s2, jnp.bfloat16),
            jnp.asarray(pa, jnp.bfloat16), jnp.asarray(m))


def _prologue_kern(x_ref, anw_ref, wq_ref, wkd_ref, wkuk_ref, wkuv_ref,
                   c2_ref, s2_ref, pa_ref, m_ref, qw_ref, kw_ref,
                   q_out, k_out, v_out):
    x = x_ref[...]
    var = jnp.mean(x * x, axis=1, keepdims=True)
    xn = x * lax.rsqrt(var + EPS) * anw_ref[...]
    m = m_ref[...]
    pa = pa_ref[...]
    c2 = c2_ref[...]
    s2 = s2_ref[...]

    def headnorm_rot(z, w_row):
        ssq = jnp.dot(z * z, m, preferred_element_type=jnp.float32) / HEAD_DIM
        rs = lax.rsqrt(ssq + EPS)
        bc = lax.dot_general(rs, m, (((1,), (1,)), ((), ())),
                             preferred_element_type=jnp.float32)
        zn = z * bc * w_row
        return (zn * c2.astype(jnp.float32)
                + jnp.dot(zn.astype(jnp.bfloat16), pa,
                          preferred_element_type=jnp.float32)
                * s2.astype(jnp.float32))

    xnb = xn.astype(jnp.bfloat16)
    q0 = jnp.dot(xnb, wq_ref[...].astype(jnp.bfloat16),
                 preferred_element_type=jnp.float32)
    qr = headnorm_rot(q0, qw_ref[...])
    lat = jnp.dot(xnb, wkd_ref[...].astype(jnp.bfloat16),
                  preferred_element_type=jnp.float32)
    latb = lat.astype(jnp.bfloat16)
    k0 = jnp.dot(latb, wkuk_ref[...].astype(jnp.bfloat16),
                 preferred_element_type=jnp.float32)
    kr = headnorm_rot(k0, kw_ref[...])
    vr = jnp.dot(latb, wkuv_ref[...].astype(jnp.bfloat16),
                 preferred_element_type=jnp.float32)
    for h in range(N_HEADS):
        sl = slice(h * HEAD_DIM, (h + 1) * HEAD_DIM)
        q_out[h] = qr[:, sl]
        k_out[h] = kr[:, sl]
        v_out[h] = vr[:, sl]


def _attn_kern(q_ref, k_ref, v_ref, o_ref):
    qt = pl.program_id(1)
    q = (q_ref[0] * (1.0 / np.sqrt(HEAD_DIM).astype(np.float32))
         ).astype(jnp.bfloat16)
    s = lax.dot_general(q, k_ref[0].astype(jnp.bfloat16),
                        (((1,), (1,)), ((), ())),
                        preferred_element_type=jnp.float32)
    row = qt * TA + lax.broadcasted_iota(jnp.int32, (TA, T), 0)
    col = lax.broadcasted_iota(jnp.int32, (TA, T), 1)
    s = jnp.where(col <= row, s, -1e9)
    mx = jnp.max(s, axis=1, keepdims=True)
    p = jnp.exp(s - mx)
    l = jnp.sum(p, axis=1, keepdims=True)
    o = jnp.dot(p.astype(jnp.bfloat16), v_ref[0].astype(jnp.bfloat16),
                preferred_element_type=jnp.float32)
    o_ref[0] = o / l


def _oproj_shared_kern(a_ref, wo_ref, x_ref, fw_ref, w1_ref, w2_ref,
                       hn_out, base_out):
    a = jnp.concatenate([a_ref[i] for i in range(N_HEADS)], axis=1)
    h = x_ref[...] + jnp.dot(a.astype(jnp.bfloat16), wo_ref[...],
                             preferred_element_type=jnp.float32)
    var = jnp.mean(h * h, axis=1, keepdims=True)
    hn = h * lax.rsqrt(var + EPS) * fw_ref[...]
    hn_out[...] = hn
    sh = jnp.dot(hn.astype(jnp.bfloat16), w1_ref[...],
                 preferred_element_type=jnp.float32)
    sh = sh / (1.0 + jnp.exp(-sh))
    base_out[...] = h + jnp.dot(sh.astype(jnp.bfloat16), w2_ref[...],
                                preferred_element_type=jnp.float32)


def _gate_kern(hn_ref, gw_ref, aux_ref, w0_ref, w1_ref,
               pos0_ref, pos1_ref, poff_ref, cnt_ref):
    hn = hn_ref[...]
    logits = jnp.dot(hn, gw_ref[...], preferred_element_type=jnp.float32)
    mx = jnp.max(logits, axis=1, keepdims=True)
    ex = jnp.exp(logits - mx)
    probs = ex / jnp.sum(ex, axis=1, keepdims=True)
    auxv = jnp.sum(jnp.mean(probs, axis=0) * jnp.mean(logits, axis=0)
                   ) * NUM_EXPERTS
    aux_ref[...] = jnp.broadcast_to(auxv, (1, 1))

    iE = lax.broadcasted_iota(jnp.int32, (T, NUM_EXPERTS), 1)
    big = jnp.int32(NUM_EXPERTS)
    i1 = jnp.min(jnp.where(logits == mx, iE, big), axis=1, keepdims=True)
    oh1 = iE == i1
    masked = jnp.where(oh1, -jnp.inf, logits)
    m2 = jnp.max(masked, axis=1, keepdims=True)
    i2 = jnp.min(jnp.where(masked == m2, iE, big), axis=1, keepdims=True)
    oh2 = iE == i2
    p1 = jnp.sum(jnp.where(oh1, probs, 0.0), axis=1, keepdims=True)
    p2 = jnp.sum(jnp.where(oh2, probs, 0.0), axis=1, keepdims=True)
    denom = p1 + p2
    w0_ref[...] = jnp.broadcast_to(p1 / denom, (T, 16))
    w1_ref[...] = jnp.broadcast_to(p2 / denom, (T, 16))

    o1f = oh1.astype(jnp.float32)
    o2f = oh2.astype(jnp.float32)
    cnt = jnp.sum(o1f, axis=0, keepdims=True) + jnp.sum(o2f, axis=0,
                                                        keepdims=True)
    cnt_i = cnt.astype(jnp.int32)
    pcnt_i = ((cnt_i + 7) // 8) * 8
    pcnt = pcnt_i.astype(jnp.float32)
    er = lax.broadcasted_iota(jnp.int32, (NUM_EXPERTS, NUM_EXPERTS), 0)
    ec = lax.broadcasted_iota(jnp.int32, (NUM_EXPERTS, NUM_EXPERTS), 1)
    upper = (er < ec).astype(jnp.float32)
    poff = jnp.dot(pcnt, upper, preferred_element_type=jnp.float32)

    rr = lax.broadcasted_iota(jnp.int32, (CH, CH), 0)
    rc = lax.broadcasted_iota(jnp.int32, (CH, CH), 1)
    lstrict = (rc < rr).astype(jnp.float32)

    def ranks(ohf, carry):
        chunks = []
        for i in range(T // CH):
            blk = ohf[i * CH:(i + 1) * CH]
            chunks.append(jnp.dot(lstrict, blk,
                                  preferred_element_type=jnp.float32) + carry)
            carry = carry + jnp.sum(blk, axis=0, keepdims=True)
        return jnp.concatenate(chunks, axis=0), carry

    rank0, carry = ranks(o1f, jnp.zeros((1, NUM_EXPERTS), jnp.float32))
    rank1, _ = ranks(o2f, carry)
    pos0 = jnp.sum(jnp.where(oh1, poff + rank0, 0.0), axis=1, keepdims=True)
    pos1 = jnp.sum(jnp.where(oh2, poff + rank1, 0.0), axis=1, keepdims=True)
    pos0_ref[...] = pos0.astype(jnp.int32)
    pos1_ref[...] = pos1.astype(jnp.int32)
    poff_ref[...] = poff.astype(jnp.int32)
    cnt_ref[...] = cnt_i


def _expert_kern(poff_ref, cnt_ref, g_ref, w1_ref, w2_ref, y_ref):
    e = pl.program_id(0)
    off = poff_ref[0, e]
    c = cnt_ref[0, e]
    nt = (c + MT - 1) // MT
    w1 = w1_ref[0].astype(jnp.bfloat16)
    w2 = w2_ref[0].astype(jnp.bfloat16)

    def body(j, _):
        base = pl.multiple_of(off + j * MT, 8)
        rows = g_ref[pl.ds(base, MT), :].astype(jnp.bfloat16)
        a = jnp.dot(rows, w1, preferred_element_type=jnp.float32)
        a = a / (1.0 + jnp.exp(-a))
        y_ref[pl.ds(base, MT), :] = jnp.dot(a.astype(jnp.bfloat16), w2,
                                            preferred_element_type=jnp.float32)
        return 0

    lax.fori_loop(0, nt, body, 0)


@functools.cache
def _sc_mesh():
    return plsc.VectorSubcoreMesh(core_axis_name="c", subcore_axis_name="s",
                                  num_cores=2, num_subcores=16)


def _dispatch_sc(hn, pos0d, pos1d):
    k = pl.kernel(
        _dispatch_body,
        out_type=jax.ShapeDtypeStruct((GR, DIM), jnp.float32),
        mesh=_sc_mesh(),
        scratch_types=[
            pltpu.VMEM((TPW,), jnp.int32),
            pltpu.VMEM((TPW, DIM), jnp.float32),
            pltpu.SemaphoreType.DMA,
        ],
    )
    return k(hn, pos0d, pos1d)


def _dispatch_body(hn_hbm, pos0_hbm, pos1_hbm, g_hbm, idx_v, rows_v, sem):
    w = lax.axis_index("s") * 2 + lax.axis_index("c")
    base = w * TPW
    pltpu.sync_copy(hn_hbm.at[pl.ds(base, TPW)], rows_v)
    pltpu.sync_copy(pos0_hbm.at[w], idx_v)
    pltpu.async_copy(rows_v, g_hbm.at[idx_v], sem).wait()
    pltpu.sync_copy(pos1_hbm.at[w], idx_v)
    pltpu.async_copy(rows_v, g_hbm.at[idx_v], sem).wait()


def _combine_sc(base, y, pos0c, pos1c, w0e, w1e):
    k = pl.kernel(
        _combine_body,
        out_type=jax.ShapeDtypeStruct((T, DIM), jnp.float32),
        mesh=_sc_mesh(),
        scratch_types=[
            pltpu.VMEM((SB,), jnp.int32),
            pltpu.VMEM((SB,), jnp.int32),
            pltpu.VMEM((SB, DIM), jnp.float32),
            pltpu.VMEM((SB, DIM), jnp.float32),
            pltpu.VMEM((SB, DIM), jnp.float32),
            pltpu.VMEM((SB, 16), jnp.float32),
            pltpu.VMEM((SB, 16), jnp.float32),
            pltpu.SemaphoreType.DMA,
        ],
    )
    return k(base, y, pos0c, pos1c, w0e, w1e)


def _combine_body(base_hbm, y_hbm, pos0_hbm, pos1_hbm, w0_hbm, w1_hbm, out_hbm,
                  idx0_v, idx1_v, y0_v, y1_v, acc_v, w0_v, w1_v, sem):
    w = lax.axis_index("s") * 2 + lax.axis_index("c")
    for sb in range(TPW // SB):
        tok0 = w * TPW + sb * SB
        ci0 = pltpu.async_copy(pos0_hbm.at[w, sb], idx0_v, sem)
        ci1 = pltpu.async_copy(pos1_hbm.at[w, sb], idx1_v, sem)
        ci0.wait()
        ci1.wait()
        c0 = pltpu.async_copy(y_hbm.at[idx0_v], y0_v, sem)
        c1 = pltpu.async_copy(y_hbm.at[idx1_v], y1_v, sem)
        c2 = pltpu.async_copy(base_hbm.at[pl.ds(tok0, SB)], acc_v, sem)
        c3 = pltpu.async_copy(w0_hbm.at[pl.ds(tok0, SB)], w0_v, sem)
        c4 = pltpu.async_copy(w1_hbm.at[pl.ds(tok0, SB)], w1_v, sem)
        c0.wait()
        c1.wait()
        c2.wait()
        c3.wait()
        c4.wait()

        def row_body(r, _):
            w0s = w0_v[r]
            w1s = w1_v[r]
            for cch in range(DIM // 16):
                sl = pl.ds(cch * 16, 16)
                acc_v[r, sl] = (acc_v[r, sl] + w0s * y0_v[r, sl]
                                + w1s * y1_v[r, sl])
            return 0

        lax.fori_loop(0, SB, row_body, 0)
        pltpu.sync_copy(acc_v, out_hbm.at[pl.ds(tok0, SB)])


def _full(shape):
    return pl.BlockSpec(shape, lambda *_: tuple(0 for _ in shape))


def _row(dim):
    return pl.BlockSpec((1, dim), lambda *_: (0, 0))


def kernel(x, attn_norm_w, wq, w_kv_down, w_kv_up, wo, q_norm_w, k_norm_w,
           ffn_norm_w, gate_w, shared_w1, shared_w2, routed_w1, routed_w2):
    xf = x.reshape(T, DIM)
    c2, s2, pa, m = _rotary_consts()
    wku = w_kv_up.reshape(KV_LORA, 2, N_HEADS * HEAD_DIM)
    wkuk = wku[:, 0]
    wkuv = wku[:, 1]
    qw = jnp.tile(q_norm_w, N_HEADS).reshape(1, DIM)
    kw = jnp.tile(k_norm_w, N_HEADS).reshape(1, DIM)
    anw = attn_norm_w.reshape(1, DIM)
    fw = ffn_norm_w.reshape(1, DIM)

    tile = pl.BlockSpec((TB, DIM), lambda t: (t, 0))
    q, k, v = pl.pallas_call(
        _prologue_kern,
        grid=(T // TB,),
        in_specs=[tile, _row(DIM), _full((DIM, DIM)), _full((DIM, KV_LORA)),
                  _full((KV_LORA, DIM)), _full((KV_LORA, DIM)), tile, tile,
                  _full((DIM, DIM)), _full((DIM, N_HEADS)), _row(DIM),
                  _row(DIM)],
        out_specs=[pl.BlockSpec((N_HEADS, TB, HEAD_DIM),
                                lambda t: (0, t, 0))] * 3,
        out_shape=[jax.ShapeDtypeStruct((N_HEADS, T, HEAD_DIM),
                                        jnp.float32)] * 3,
    )(xf, anw, wq, w_kv_down, wkuk, wkuv, c2, s2, pa, m, qw, kw)

    attn = pl.pallas_call(
        _attn_kern,
        grid=(N_HEADS, T // TA),
        in_specs=[pl.BlockSpec((1, TA, HEAD_DIM), lambda h, t: (h, t, 0)),
                  pl.BlockSpec((1, T, HEAD_DIM), lambda h, t: (h, 0, 0)),
                  pl.BlockSpec((1, T, HEAD_DIM), lambda h, t: (h, 0, 0))],
        out_specs=pl.BlockSpec((1, TA, HEAD_DIM), lambda h, t: (h, t, 0)),
        out_shape=jax.ShapeDtypeStruct((N_HEADS, T, HEAD_DIM), jnp.float32),
    )(q, k, v)

    w1cat = jnp.concatenate([shared_w1[0], shared_w1[1]],
                            axis=1).astype(jnp.bfloat16)
    w2cat = jnp.concatenate([shared_w2[0], shared_w2[1]],
                            axis=0).astype(jnp.bfloat16)
    hn, base = pl.pallas_call(
        _oproj_shared_kern,
        grid=(T // TB,),
        in_specs=[pl.BlockSpec((N_HEADS, TB, HEAD_DIM),
                               lambda t: (0, t, 0)),
                  _full((DIM, DIM)), tile, _row(DIM),
                  _full((DIM, 2 * EXPERT_HIDDEN)),
                  _full((2 * EXPERT_HIDDEN, DIM))],
        out_specs=[tile, tile],
        out_shape=[jax.ShapeDtypeStruct((T, DIM), jnp.float32)] * 2,
    )(attn, wo.astype(jnp.bfloat16), xf, fw, w1cat, w2cat)

    aux, w0e, w1e, pos0, pos1, poff, cnt = pl.pallas_call(
        _gate_kern,
        grid=(1,),
        in_specs=[_full((T, DIM)), _full((DIM, NUM_EXPERTS))],
        out_specs=[_full((1, 1)), _full((T, 16)), _full((T, 16)),
                   _full((T, 1)), _full((T, 1)), _full((1, NUM_EXPERTS)),
                   _full((1, NUM_EXPERTS))],
        out_shape=[
            jax.ShapeDtypeStruct((1, 1), jnp.float32),
            jax.ShapeDtypeStruct((T, 16), jnp.float32),
            jax.ShapeDtypeStruct((T, 16), jnp.float32),
            jax.ShapeDtypeStruct((T, 1), jnp.int32),
            jax.ShapeDtypeStruct((T, 1), jnp.int32),
            jax.ShapeDtypeStruct((1, NUM_EXPERTS), jnp.int32),
            jax.ShapeDtypeStruct((1, NUM_EXPERTS), jnp.int32),
        ],
    )(hn, gate_w)

    pos0d = pos0.reshape(NW, TPW)
    pos1d = pos1.reshape(NW, TPW)
    g = _dispatch_sc(hn, pos0d, pos1d)

    y = pl.pallas_call(
        _expert_kern,
        grid=(NUM_EXPERTS,),
        in_specs=[pl.BlockSpec(memory_space=pltpu.SMEM),
                  pl.BlockSpec(memory_space=pltpu.SMEM),
                  _full((GR, DIM)),
                  pl.BlockSpec((1, DIM, EXPERT_HIDDEN), lambda e: (e, 0, 0)),
                  pl.BlockSpec((1, EXPERT_HIDDEN, DIM), lambda e: (e, 0, 0))],
        out_specs=_full((GR, DIM)),
        out_shape=jax.ShapeDtypeStruct((GR, DIM), jnp.float32),
    )(poff, cnt, g, routed_w1, routed_w2)

    pos0c = pos0.reshape(NW, TPW // SB, SB)
    pos1c = pos1.reshape(NW, TPW // SB, SB)
    out = _combine_sc(base, y, pos0c, pos1c, w0e, w1e)

    return out.reshape(x.shape), aux.reshape(())


# speedup vs baseline: 1.1770x; 1.0092x over previous
import functools

import numpy as np
import jax
import jax.numpy as jnp
from jax import lax
from jax.experimental import pallas as pl
from jax.experimental.pallas import tpu as pltpu
from jax.experimental.pallas import tpu_sc as plsc

DIM = 768
N_HEADS = 12
HEAD_DIM = 64
KV_LORA = 256
NUM_EXPERTS = 64
EXPERT_HIDDEN = 256
T = 2048
EPS = 1e-6
TB = 256
MT = 128
NW = 32
TPW = T // NW
SB = 32
GR = 4800
CH = 128
TA = 256


def _rotary_consts():
    freqs = 1.0 / (10000.0 ** (np.arange(0, HEAD_DIM, 2)[: HEAD_DIM // 2]
                               .astype(np.float32) / HEAD_DIM))
    t = np.arange(T, dtype=np.float32)
    f = np.outer(t, freqs)
    cos, sin = np.cos(f), np.sin(f)
    c_rep = np.repeat(cos, 2, axis=1)
    s_rep = np.repeat(sin, 2, axis=1)
    sgn = np.tile(np.array([-1.0, 1.0], np.float32), HEAD_DIM // 2)
    c2 = np.tile(c_rep, (1, N_HEADS))
    s2 = np.tile(s_rep * sgn[None, :], (1, N_HEADS))
    pa = np.zeros((DIM, DIM), np.float32)
    idx = np.arange(DIM)
    swapped = idx ^ 1
    pa[swapped, idx] = 1.0
    m = np.zeros((DIM, N_HEADS), np.float32)
    for h in range(N_HEADS):
        m[h * HEAD_DIM:(h + 1) * HEAD_DIM, h] = 1.0
    return (jnp.asarray(c2, jnp.bfloat16), jnp.asarray(s2, jnp.bfloat16),
            jnp.asarray(pa, jnp.bfloat16), jnp.asarray(m))


def _prologue_kern(x_ref, anw_ref, wq_ref, wkd_ref, wkuk_ref, wkuv_ref,
                   c2_ref, s2_ref, pa_ref, m_ref, qw_ref, kw_ref,
                   q_out, k_out, v_out):
    x = x_ref[...]
    var = jnp.mean(x * x, axis=1, keepdims=True)
    xn = x * lax.rsqrt(var + EPS) * anw_ref[...]
    m = m_ref[...]
    pa = pa_ref[...]
    c2 = c2_ref[...]
    s2 = s2_ref[...]

    def headnorm_rot(z, w_row):
        ssq = jnp.dot(z * z, m, preferred_element_type=jnp.float32) / HEAD_DIM
        rs = lax.rsqrt(ssq + EPS)
        bc = lax.dot_general(rs, m, (((1,), (1,)), ((), ())),
                             preferred_element_type=jnp.float32)
        zn = z * bc * w_row
        return (zn * c2.astype(jnp.float32)
                + jnp.dot(zn.astype(jnp.bfloat16), pa,
                          preferred_element_type=jnp.float32)
                * s2.astype(jnp.float32))

    xnb = xn.astype(jnp.bfloat16)
    q0 = jnp.dot(xnb, wq_ref[...].astype(jnp.bfloat16),
                 preferred_element_type=jnp.float32)
    qr = headnorm_rot(q0, qw_ref[...])
    lat = jnp.dot(xnb, wkd_ref[...].astype(jnp.bfloat16),
                  preferred_element_type=jnp.float32)
    latb = lat.astype(jnp.bfloat16)
    k0 = jnp.dot(latb, wkuk_ref[...].astype(jnp.bfloat16),
                 preferred_element_type=jnp.float32)
    kr = headnorm_rot(k0, kw_ref[...])
    vr = jnp.dot(latb, wkuv_ref[...].astype(jnp.bfloat16),
                 preferred_element_type=jnp.float32)
    for h in range(N_HEADS):
        sl = slice(h * HEAD_DIM, (h + 1) * HEAD_DIM)
        q_out[h] = qr[:, sl]
        k_out[h] = kr[:, sl]
        v_out[h] = vr[:, sl]


def _attn_kern(q_ref, k_ref, v_ref, o_ref, *, kw, qoff):
    qt = pl.program_id(1)
    q = (q_ref[0] * (1.0 / np.sqrt(HEAD_DIM).astype(np.float32))
         ).astype(jnp.bfloat16)
    s = lax.dot_general(q, k_ref[0].astype(jnp.bfloat16),
                        (((1,), (1,)), ((), ())),
                        preferred_element_type=jnp.float32)
    row = (qt + qoff) * TA + lax.broadcasted_iota(jnp.int32, (TA, kw), 0)
    col = lax.broadcasted_iota(jnp.int32, (TA, kw), 1)
    s = jnp.where(col <= row, s, -1e9)
    mx = jnp.max(s, axis=1, keepdims=True)
    p = jnp.exp(s - mx)
    l = jnp.sum(p, axis=1, keepdims=True)
    o = jnp.dot(p.astype(jnp.bfloat16), v_ref[0].astype(jnp.bfloat16),
                preferred_element_type=jnp.float32)
    o_ref[0] = o / l


def _oproj_shared_kern(a_ref, wo_ref, x_ref, fw_ref, w1_ref, w2_ref,
                       hn_out, base_out):
    a = jnp.concatenate([a_ref[i] for i in range(N_HEADS)], axis=1)
    h = x_ref[...] + jnp.dot(a.astype(jnp.bfloat16), wo_ref[...],
                             preferred_element_type=jnp.float32)
    var = jnp.mean(h * h, axis=1, keepdims=True)
    hn = h * lax.rsqrt(var + EPS) * fw_ref[...]
    hn_out[...] = hn
    sh = jnp.dot(hn.astype(jnp.bfloat16), w1_ref[...],
                 preferred_element_type=jnp.float32)
    sh = sh / (1.0 + jnp.exp(-sh))
    base_out[...] = h + jnp.dot(sh.astype(jnp.bfloat16), w2_ref[...],
                                preferred_element_type=jnp.float32)


def _gate_kern(hn_ref, gw_ref, aux_ref, w0_ref, w1_ref,
               pos0_ref, pos1_ref, poff_ref, cnt_ref):
    hn = hn_ref[...]
    logits = jnp.dot(hn, gw_ref[...], preferred_element_type=jnp.float32)
    mx = jnp.max(logits, axis=1, keepdims=True)
    ex = jnp.exp(logits - mx)
    probs = ex / jnp.sum(ex, axis=1, keepdims=True)
    auxv = jnp.sum(jnp.mean(probs, axis=0) * jnp.mean(logits, axis=0)
                   ) * NUM_EXPERTS
    aux_ref[...] = jnp.broadcast_to(auxv, (1, 1))

    iE = lax.broadcasted_iota(jnp.int32, (T, NUM_EXPERTS), 1)
    big = jnp.int32(NUM_EXPERTS)
    i1 = jnp.min(jnp.where(logits == mx, iE, big), axis=1, keepdims=True)
    oh1 = iE == i1
    masked = jnp.where(oh1, -jnp.inf, logits)
    m2 = jnp.max(masked, axis=1, keepdims=True)
    i2 = jnp.min(jnp.where(masked == m2, iE, big), axis=1, keepdims=True)
    oh2 = iE == i2
    p1 = jnp.sum(jnp.where(oh1, probs, 0.0), axis=1, keepdims=True)
    p2 = jnp.sum(jnp.where(oh2, probs, 0.0), axis=1, keepdims=True)
    denom = p1 + p2
    w0_ref[...] = jnp.broadcast_to(p1 / denom, (T, 16))
    w1_ref[...] = jnp.broadcast_to(p2 / denom, (T, 16))

    o1f = oh1.astype(jnp.float32)
    o2f = oh2.astype(jnp.float32)
    cnt = jnp.sum(o1f, axis=0, keepdims=True) + jnp.sum(o2f, axis=0,
                                                        keepdims=True)
    cnt_i = cnt.astype(jnp.int32)
    pcnt_i = ((cnt_i + 7) // 8) * 8
    pcnt = pcnt_i.astype(jnp.float32)
    er = lax.broadcasted_iota(jnp.int32, (NUM_EXPERTS, NUM_EXPERTS), 0)
    ec = lax.broadcasted_iota(jnp.int32, (NUM_EXPERTS, NUM_EXPERTS), 1)
    upper = (er < ec).astype(jnp.float32)
    poff = jnp.dot(pcnt, upper, preferred_element_type=jnp.float32)

    rr = lax.broadcasted_iota(jnp.int32, (CH, CH), 0)
    rc = lax.broadcasted_iota(jnp.int32, (CH, CH), 1)
    lstrict = (rc < rr).astype(jnp.float32)

    def ranks(ohf, carry):
        chunks = []
        for i in range(T // CH):
            blk = ohf[i * CH:(i + 1) * CH]
            chunks.append(jnp.dot(lstrict, blk,
                                  preferred_element_type=jnp.float32) + carry)
            carry = carry + jnp.sum(blk, axis=0, keepdims=True)
        return jnp.concatenate(chunks, axis=0), carry

    rank0, carry = ranks(o1f, jnp.zeros((1, NUM_EXPERTS), jnp.float32))
    rank1, _ = ranks(o2f, carry)
    pos0 = jnp.sum(jnp.where(oh1, poff + rank0, 0.0), axis=1, keepdims=True)
    pos1 = jnp.sum(jnp.where(oh2, poff + rank1, 0.0), axis=1, keepdims=True)
    pos0_ref[...] = pos0.astype(jnp.int32)
    pos1_ref[...] = pos1.astype(jnp.int32)
    poff_ref[...] = poff.astype(jnp.int32)
    cnt_ref[...] = cnt_i


def _expert_kern(poff_ref, cnt_ref, g_ref, w1_ref, w2_ref, y_ref):
    e = pl.program_id(0)
    off = poff_ref[0, e]
    c = cnt_ref[0, e]
    nt = (c + MT - 1) // MT
    w1 = w1_ref[0].astype(jnp.bfloat16)
    w2 = w2_ref[0].astype(jnp.bfloat16)

    def body(j, _):
        base = pl.multiple_of(off + j * MT, 8)
        rows = g_ref[pl.ds(base, MT), :].astype(jnp.bfloat16)
        a = jnp.dot(rows, w1, preferred_element_type=jnp.float32)
        a = a / (1.0 + jnp.exp(-a))
        y_ref[pl.ds(base, MT), :] = jnp.dot(a.astype(jnp.bfloat16), w2,
                                            preferred_element_type=jnp.float32)
        return 0

    lax.fori_loop(0, nt, body, 0)


@functools.cache
def _sc_mesh():
    return plsc.VectorSubcoreMesh(core_axis_name="c", subcore_axis_name="s",
                                  num_cores=2, num_subcores=16)


def _dispatch_sc(hn, pos0d, pos1d):
    k = pl.kernel(
        _dispatch_body,
        out_type=jax.ShapeDtypeStruct((GR, DIM), jnp.float32),
        mesh=_sc_mesh(),
        scratch_types=[
            pltpu.VMEM((TPW,), jnp.int32),
            pltpu.VMEM((TPW, DIM), jnp.float32),
            pltpu.SemaphoreType.DMA,
        ],
    )
    return k(hn, pos0d, pos1d)


def _dispatch_body(hn_hbm, pos0_hbm, pos1_hbm, g_hbm, idx_v, rows_v, sem):
    w = lax.axis_index("s") * 2 + lax.axis_index("c")
    base = w * TPW
    pltpu.sync_copy(hn_hbm.at[pl.ds(base, TPW)], rows_v)
    pltpu.sync_copy(pos0_hbm.at[w], idx_v)
    pltpu.async_copy(rows_v, g_hbm.at[idx_v], sem).wait()
    pltpu.sync_copy(pos1_hbm.at[w], idx_v)
    pltpu.async_copy(rows_v, g_hbm.at[idx_v], sem).wait()


def _combine_sc(base, y, pos0c, pos1c, w0e, w1e):
    k = pl.kernel(
        _combine_body,
        out_type=jax.ShapeDtypeStruct((T, DIM), jnp.float32),
        mesh=_sc_mesh(),
        scratch_types=[
            pltpu.VMEM((SB,), jnp.int32),
            pltpu.VMEM((SB,), jnp.int32),
            pltpu.VMEM((SB, DIM), jnp.float32),
            pltpu.VMEM((SB, DIM), jnp.float32),
            pltpu.VMEM((SB, DIM), jnp.float32),
            pltpu.VMEM((SB, 16), jnp.float32),
            pltpu.VMEM((SB, 16), jnp.float32),
            pltpu.SemaphoreType.DMA,
        ],
    )
    return k(base, y, pos0c, pos1c, w0e, w1e)


def _combine_body(base_hbm, y_hbm, pos0_hbm, pos1_hbm, w0_hbm, w1_hbm, out_hbm,
                  idx0_v, idx1_v, y0_v, y1_v, acc_v, w0_v, w1_v, sem):
    w = lax.axis_index("s") * 2 + lax.axis_index("c")
    for sb in range(TPW // SB):
        tok0 = w * TPW + sb * SB
        ci0 = pltpu.async_copy(pos0_hbm.at[w, sb], idx0_v, sem)
        ci1 = pltpu.async_copy(pos1_hbm.at[w, sb], idx1_v, sem)
        ci0.wait()
        ci1.wait()
        c0 = pltpu.async_copy(y_hbm.at[idx0_v], y0_v, sem)
        c1 = pltpu.async_copy(y_hbm.at[idx1_v], y1_v, sem)
        c2 = pltpu.async_copy(base_hbm.at[pl.ds(tok0, SB)], acc_v, sem)
        c3 = pltpu.async_copy(w0_hbm.at[pl.ds(tok0, SB)], w0_v, sem)
        c4 = pltpu.async_copy(w1_hbm.at[pl.ds(tok0, SB)], w1_v, sem)
        c0.wait()
        c1.wait()
        c2.wait()
        c3.wait()
        c4.wait()

        def row_body(r, _):
            w0s = w0_v[r]
            w1s = w1_v[r]
            for cch in range(DIM // 16):
                sl = pl.ds(cch * 16, 16)
                acc_v[r, sl] = (acc_v[r, sl] + w0s * y0_v[r, sl]
                                + w1s * y1_v[r, sl])
            return 0

        lax.fori_loop(0, SB, row_body, 0)
        pltpu.sync_copy(acc_v, out_hbm.at[pl.ds(tok0, SB)])


def _full(shape):
    return pl.BlockSpec(shape, lambda *_: tuple(0 for _ in shape))


def _row(dim):
    return pl.BlockSpec((1, dim), lambda *_: (0, 0))


def kernel(x, attn_norm_w, wq, w_kv_down, w_kv_up, wo, q_norm_w, k_norm_w,
           ffn_norm_w, gate_w, shared_w1, shared_w2, routed_w1, routed_w2):
    xf = x.reshape(T, DIM)
    c2, s2, pa, m = _rotary_consts()
    wku = w_kv_up.reshape(KV_LORA, 2, N_HEADS * HEAD_DIM)
    wkuk = wku[:, 0]
    wkuv = wku[:, 1]
    qw = jnp.tile(q_norm_w, N_HEADS).reshape(1, DIM)
    kw = jnp.tile(k_norm_w, N_HEADS).reshape(1, DIM)
    anw = attn_norm_w.reshape(1, DIM)
    fw = ffn_norm_w.reshape(1, DIM)

    tile = pl.BlockSpec((TB, DIM), lambda t: (t, 0))
    q, k, v = pl.pallas_call(
        _prologue_kern,
        grid=(T // TB,),
        in_specs=[tile, _row(DIM), _full((DIM, DIM)), _full((DIM, KV_LORA)),
                  _full((KV_LORA, DIM)), _full((KV_LORA, DIM)), tile, tile,
                  _full((DIM, DIM)), _full((DIM, N_HEADS)), _row(DIM),
                  _row(DIM)],
        out_specs=[pl.BlockSpec((N_HEADS, TB, HEAD_DIM),
                                lambda t: (0, t, 0))] * 3,
        out_shape=[jax.ShapeDtypeStruct((N_HEADS, T, HEAD_DIM),
                                        jnp.float32)] * 3,
    )(xf, anw, wq, w_kv_down, wkuk, wkuv, c2, s2, pa, m, qw, kw)

    half = T // 2
    attn_lo = pl.pallas_call(
        functools.partial(_attn_kern, kw=half, qoff=0),
        grid=(N_HEADS, half // TA),
        in_specs=[pl.BlockSpec((1, TA, HEAD_DIM), lambda h, t: (h, t, 0)),
                  pl.BlockSpec((1, half, HEAD_DIM), lambda h, t: (h, 0, 0)),
                  pl.BlockSpec((1, half, HEAD_DIM), lambda h, t: (h, 0, 0))],
        out_specs=pl.BlockSpec((1, TA, HEAD_DIM), lambda h, t: (h, t, 0)),
        out_shape=jax.ShapeDtypeStruct((N_HEADS, half, HEAD_DIM),
                                       jnp.float32),
    )(q, k, v)
    attn_hi = pl.pallas_call(
        functools.partial(_attn_kern, kw=T, qoff=half // TA),
        grid=(N_HEADS, half // TA),
        in_specs=[pl.BlockSpec((1, TA, HEAD_DIM),
                               lambda h, t: (h, t + half // TA, 0)),
                  pl.BlockSpec((1, T, HEAD_DIM), lambda h, t: (h, 0, 0)),
                  pl.BlockSpec((1, T, HEAD_DIM), lambda h, t: (h, 0, 0))],
        out_specs=pl.BlockSpec((1, TA, HEAD_DIM), lambda h, t: (h, t, 0)),
        out_shape=jax.ShapeDtypeStruct((N_HEADS, half, HEAD_DIM),
                                       jnp.float32),
    )(q, k, v)
    attn = jnp.concatenate([attn_lo, attn_hi], axis=1)
    w1cat = jnp.concatenate([shared_w1[0], shared_w1[1]],
                            axis=1).astype(jnp.bfloat16)
    w2cat = jnp.concatenate([shared_w2[0], shared_w2[1]],
                            axis=0).astype(jnp.bfloat16)
    hn, base = pl.pallas_call(
        _oproj_shared_kern,
        grid=(T // TB,),
        in_specs=[pl.BlockSpec((N_HEADS, TB, HEAD_DIM),
                               lambda t: (0, t, 0)),
                  _full((DIM, DIM)), tile, _row(DIM),
                  _full((DIM, 2 * EXPERT_HIDDEN)),
                  _full((2 * EXPERT_HIDDEN, DIM))],
        out_specs=[tile, tile],
        out_shape=[jax.ShapeDtypeStruct((T, DIM), jnp.float32)] * 2,
    )(attn, wo.astype(jnp.bfloat16), xf, fw, w1cat, w2cat)

    aux, w0e, w1e, pos0, pos1, poff, cnt = pl.pallas_call(
        _gate_kern,
        grid=(1,),
        in_specs=[_full((T, DIM)), _full((DIM, NUM_EXPERTS))],
        out_specs=[_full((1, 1)), _full((T, 16)), _full((T, 16)),
                   _full((T, 1)), _full((T, 1)), _full((1, NUM_EXPERTS)),
                   _full((1, NUM_EXPERTS))],
        out_shape=[
            jax.ShapeDtypeStruct((1, 1), jnp.float32),
            jax.ShapeDtypeStruct((T, 16), jnp.float32),
            jax.ShapeDtypeStruct((T, 16), jnp.float32),
            jax.ShapeDtypeStruct((T, 1), jnp.int32),
            jax.ShapeDtypeStruct((T, 1), jnp.int32),
            jax.ShapeDtypeStruct((1, NUM_EXPERTS), jnp.int32),
            jax.ShapeDtypeStruct((1, NUM_EXPERTS), jnp.int32),
        ],
    )(hn, gate_w)

    pos0d = pos0.reshape(NW, TPW)
    pos1d = pos1.reshape(NW, TPW)
    g = _dispatch_sc(hn, pos0d, pos1d)

    y = pl.pallas_call(
        _expert_kern,
        grid=(NUM_EXPERTS,),
        in_specs=[pl.BlockSpec(memory_space=pltpu.SMEM),
                  pl.BlockSpec(memory_space=pltpu.SMEM),
                  _full((GR, DIM)),
                  pl.BlockSpec((1, DIM, EXPERT_HIDDEN), lambda e: (e, 0, 0)),
                  pl.BlockSpec((1, EXPERT_HIDDEN, DIM), lambda e: (e, 0, 0))],
        out_specs=_full((GR, DIM)),
        out_shape=jax.ShapeDtypeStruct((GR, DIM), jnp.float32),
    )(poff, cnt, g, routed_w1, routed_w2)

    pos0c = pos0.reshape(NW, TPW // SB, SB)
    pos1c = pos1.reshape(NW, TPW // SB, SB)
    out = _combine_sc(base, y, pos0c, pos1c, w0e, w1e)

    return out.reshape(x.shape), aux.reshape(())


# speedup vs baseline: 1.2373x; 1.0513x over previous
import functools

import numpy as np
import jax
import jax.numpy as jnp
from jax import lax
from jax.experimental import pallas as pl
from jax.experimental.pallas import tpu as pltpu
from jax.experimental.pallas import tpu_sc as plsc

DIM = 768
N_HEADS = 12
HEAD_DIM = 64
KV_LORA = 256
NUM_EXPERTS = 64
EXPERT_HIDDEN = 256
T = 2048
EPS = 1e-6
TB = 256
MT = 128
NW = 32
TPW = T // NW
SB = 32
GR = 4800
CH = 128
TA = 256


def _rotary_consts():
    freqs = 1.0 / (10000.0 ** (np.arange(0, HEAD_DIM, 2)[: HEAD_DIM // 2]
                               .astype(np.float32) / HEAD_DIM))
    t = np.arange(T, dtype=np.float32)
    f = np.outer(t, freqs)
    cos, sin = np.cos(f), np.sin(f)
    c_rep = np.repeat(cos, 2, axis=1)
    s_rep = np.repeat(sin, 2, axis=1)
    sgn = np.tile(np.array([-1.0, 1.0], np.float32), HEAD_DIM // 2)
    c2 = np.tile(c_rep, (1, N_HEADS))
    s2 = np.tile(s_rep * sgn[None, :], (1, N_HEADS))
    pa = np.zeros((DIM, DIM), np.float32)
    idx = np.arange(DIM)
    swapped = idx ^ 1
    pa[swapped, idx] = 1.0
    m = np.zeros((DIM, N_HEADS), np.float32)
    for h in range(N_HEADS):
        m[h * HEAD_DIM:(h + 1) * HEAD_DIM, h] = 1.0
    return (jnp.asarray(c2, jnp.bfloat16), jnp.asarray(s2, jnp.bfloat16),
            jnp.asarray(pa, jnp.bfloat16), jnp.asarray(m))


def _prologue_kern(x_ref, anw_ref, wq_ref, wkd_ref, wkuk_ref, wkuv_ref,
                   c2_ref, s2_ref, pa_ref, m_ref, qw_ref, kw_ref,
                   q_out, k_out, v_out):
    x = x_ref[...]
    var = jnp.mean(x * x, axis=1, keepdims=True)
    xn = x * lax.rsqrt(var + EPS) * anw_ref[...]
    m = m_ref[...]
    pa = pa_ref[...]
    c2 = c2_ref[...]
    s2 = s2_ref[...]

    def headnorm_rot(z, w_row):
        ssq = jnp.dot(z * z, m, preferred_element_type=jnp.float32) / HEAD_DIM
        rs = lax.rsqrt(ssq + EPS)
        bc = lax.dot_general(rs, m, (((1,), (1,)), ((), ())),
                             preferred_element_type=jnp.float32)
        zn = z * bc * w_row
        return (zn * c2.astype(jnp.float32)
                + jnp.dot(zn.astype(jnp.bfloat16), pa,
                          preferred_element_type=jnp.float32)
                * s2.astype(jnp.float32))

    xnb = xn.astype(jnp.bfloat16)
    q0 = jnp.dot(xnb, wq_ref[...].astype(jnp.bfloat16),
                 preferred_element_type=jnp.float32)
    qr = headnorm_rot(q0, qw_ref[...])
    lat = jnp.dot(xnb, wkd_ref[...].astype(jnp.bfloat16),
                  preferred_element_type=jnp.float32)
    latb = lat.astype(jnp.bfloat16)
    k0 = jnp.dot(latb, wkuk_ref[...].astype(jnp.bfloat16),
                 preferred_element_type=jnp.float32)
    kr = headnorm_rot(k0, kw_ref[...])
    vr = jnp.dot(latb, wkuv_ref[...].astype(jnp.bfloat16),
                 preferred_element_type=jnp.float32)
    qr = qr.astype(jnp.bfloat16)
    kr = kr.astype(jnp.bfloat16)
    vr = vr.astype(jnp.bfloat16)
    for h in range(N_HEADS):
        sl = slice(h * HEAD_DIM, (h + 1) * HEAD_DIM)
        q_out[h] = qr[:, sl]
        k_out[h] = kr[:, sl]
        v_out[h] = vr[:, sl]


def _attn_kern(q_ref, k_ref, v_ref, o_ref, *, kw, qoff):
    qt = pl.program_id(1)
    q = q_ref[0] * jnp.bfloat16(1.0 / np.sqrt(HEAD_DIM))
    s = lax.dot_general(q, k_ref[0], (((1,), (1,)), ((), ())),
                        preferred_element_type=jnp.float32)
    row = (qt + qoff) * TA + lax.broadcasted_iota(jnp.int32, (TA, kw), 0)
    col = lax.broadcasted_iota(jnp.int32, (TA, kw), 1)
    s = jnp.where(col <= row, s, -1e9)
    mx = jnp.max(s, axis=1, keepdims=True)
    p = jnp.exp(s - mx)
    l = jnp.sum(p, axis=1, keepdims=True)
    o = jnp.dot(p.astype(jnp.bfloat16), v_ref[0],
                preferred_element_type=jnp.float32)
    o_ref[0] = (o / l).astype(jnp.bfloat16)


def _oproj_shared_kern(a_ref, wo_ref, x_ref, fw_ref, w1_ref, w2_ref,
                       hn_out, base_out):
    a = jnp.concatenate([a_ref[i] for i in range(N_HEADS)], axis=1)
    h = x_ref[...] + jnp.dot(a, wo_ref[...],
                             preferred_element_type=jnp.float32)
    var = jnp.mean(h * h, axis=1, keepdims=True)
    hn = h * lax.rsqrt(var + EPS) * fw_ref[...]
    hn_out[...] = hn
    sh = jnp.dot(hn.astype(jnp.bfloat16), w1_ref[...],
                 preferred_element_type=jnp.float32)
    sh = sh / (1.0 + jnp.exp(-sh))
    base_out[...] = h + jnp.dot(sh.astype(jnp.bfloat16), w2_ref[...],
                                preferred_element_type=jnp.float32)


def _gate_kern(hn_ref, gw_ref, aux_ref, w0_ref, w1_ref,
               pos0_ref, pos1_ref, poff_ref, cnt_ref):
    hn = hn_ref[...]
    logits = jnp.dot(hn, gw_ref[...], preferred_element_type=jnp.float32)
    mx = jnp.max(logits, axis=1, keepdims=True)
    ex = jnp.exp(logits - mx)
    probs = ex / jnp.sum(ex, axis=1, keepdims=True)
    auxv = jnp.sum(jnp.mean(probs, axis=0) * jnp.mean(logits, axis=0)
                   ) * NUM_EXPERTS
    aux_ref[...] = jnp.broadcast_to(auxv, (1, 1))

    iE = lax.broadcasted_iota(jnp.int32, (T, NUM_EXPERTS), 1)
    big = jnp.int32(NUM_EXPERTS)
    i1 = jnp.min(jnp.where(logits == mx, iE, big), axis=1, keepdims=True)
    oh1 = iE == i1
    masked = jnp.where(oh1, -jnp.inf, logits)
    m2 = jnp.max(masked, axis=1, keepdims=True)
    i2 = jnp.min(jnp.where(masked == m2, iE, big), axis=1, keepdims=True)
    oh2 = iE == i2
    p1 = jnp.sum(jnp.where(oh1, probs, 0.0), axis=1, keepdims=True)
    p2 = jnp.sum(jnp.where(oh2, probs, 0.0), axis=1, keepdims=True)
    denom = p1 + p2
    w0_ref[...] = jnp.broadcast_to(p1 / denom, (T, 16))
    w1_ref[...] = jnp.broadcast_to(p2 / denom, (T, 16))

    o1f = oh1.astype(jnp.float32)
    o2f = oh2.astype(jnp.float32)
    cnt = jnp.sum(o1f, axis=0, keepdims=True) + jnp.sum(o2f, axis=0,
                                                        keepdims=True)
    cnt_i = cnt.astype(jnp.int32)
    pcnt_i = ((cnt_i + 7) // 8) * 8
    pcnt = pcnt_i.astype(jnp.float32)
    er = lax.broadcasted_iota(jnp.int32, (NUM_EXPERTS, NUM_EXPERTS), 0)
    ec = lax.broadcasted_iota(jnp.int32, (NUM_EXPERTS, NUM_EXPERTS), 1)
    upper = (er < ec).astype(jnp.float32)
    poff = jnp.dot(pcnt, upper, preferred_element_type=jnp.float32)

    rr = lax.broadcasted_iota(jnp.int32, (CH, CH), 0)
    rc = lax.broadcasted_iota(jnp.int32, (CH, CH), 1)
    lstrict = (rc < rr).astype(jnp.float32)

    def ranks(ohf, carry):
        chunks = []
        for i in range(T // CH):
            blk = ohf[i * CH:(i + 1) * CH]
            chunks.append(jnp.dot(lstrict, blk,
                                  preferred_element_type=jnp.float32) + carry)
            carry = carry + jnp.sum(blk, axis=0, keepdims=True)
        return jnp.concatenate(chunks, axis=0), carry

    rank0, carry = ranks(o1f, jnp.zeros((1, NUM_EXPERTS), jnp.float32))
    rank1, _ = ranks(o2f, carry)
    pos0 = jnp.sum(jnp.where(oh1, poff + rank0, 0.0), axis=1, keepdims=True)
    pos1 = jnp.sum(jnp.where(oh2, poff + rank1, 0.0), axis=1, keepdims=True)
    pos0_ref[...] = pos0.astype(jnp.int32)
    pos1_ref[...] = pos1.astype(jnp.int32)
    poff_ref[...] = poff.astype(jnp.int32)
    cnt_ref[...] = cnt_i


def _expert_kern(poff_ref, cnt_ref, g_ref, w1_ref, w2_ref, y_ref):
    e = pl.program_id(0)
    off = poff_ref[0, e]
    c = cnt_ref[0, e]
    nt = (c + MT - 1) // MT
    w1 = w1_ref[0].astype(jnp.bfloat16)
    w2 = w2_ref[0].astype(jnp.bfloat16)

    def body(j, _):
        base = pl.multiple_of(off + j * MT, 8)
        rows = g_ref[pl.ds(base, MT), :].astype(jnp.bfloat16)
        a = jnp.dot(rows, w1, preferred_element_type=jnp.float32)
        a = a / (1.0 + jnp.exp(-a))
        y_ref[pl.ds(base, MT), :] = jnp.dot(a.astype(jnp.bfloat16), w2,
                                            preferred_element_type=jnp.float32)
        return 0

    lax.fori_loop(0, nt, body, 0)


@functools.cache
def _sc_mesh():
    return plsc.VectorSubcoreMesh(core_axis_name="c", subcore_axis_name="s",
                                  num_cores=2, num_subcores=16)


def _dispatch_sc(hn, pos0d, pos1d):
    k = pl.kernel(
        _dispatch_body,
        out_type=jax.ShapeDtypeStruct((GR, DIM), jnp.float32),
        mesh=_sc_mesh(),
        scratch_types=[
            pltpu.VMEM((TPW,), jnp.int32),
            pltpu.VMEM((TPW, DIM), jnp.float32),
            pltpu.SemaphoreType.DMA,
        ],
    )
    return k(hn, pos0d, pos1d)


def _dispatch_body(hn_hbm, pos0_hbm, pos1_hbm, g_hbm, idx_v, rows_v, sem):
    w = lax.axis_index("s") * 2 + lax.axis_index("c")
    base = w * TPW
    pltpu.sync_copy(hn_hbm.at[pl.ds(base, TPW)], rows_v)
    pltpu.sync_copy(pos0_hbm.at[w], idx_v)
    pltpu.async_copy(rows_v, g_hbm.at[idx_v], sem).wait()
    pltpu.sync_copy(pos1_hbm.at[w], idx_v)
    pltpu.async_copy(rows_v, g_hbm.at[idx_v], sem).wait()


def _combine_sc(base, y, pos0c, pos1c, w0e, w1e):
    k = pl.kernel(
        _combine_body,
        out_type=jax.ShapeDtypeStruct((T, DIM), jnp.float32),
        mesh=_sc_mesh(),
        scratch_types=[
            pltpu.VMEM((SB,), jnp.int32),
            pltpu.VMEM((SB,), jnp.int32),
            pltpu.VMEM((SB, DIM), jnp.float32),
            pltpu.VMEM((SB, DIM), jnp.float32),
            pltpu.VMEM((SB, DIM), jnp.float32),
            pltpu.VMEM((SB, 16), jnp.float32),
            pltpu.VMEM((SB, 16), jnp.float32),
            pltpu.SemaphoreType.DMA,
        ],
    )
    return k(base, y, pos0c, pos1c, w0e, w1e)


def _combine_body(base_hbm, y_hbm, pos0_hbm, pos1_hbm, w0_hbm, w1_hbm, out_hbm,
                  idx0_v, idx1_v, y0_v, y1_v, acc_v, w0_v, w1_v, sem):
    w = lax.axis_index("s") * 2 + lax.axis_index("c")
    for sb in range(TPW // SB):
        tok0 = w * TPW + sb * SB
        ci0 = pltpu.async_copy(pos0_hbm.at[w, sb], idx0_v, sem)
        ci1 = pltpu.async_copy(pos1_hbm.at[w, sb], idx1_v, sem)
        ci0.wait()
        ci1.wait()
        c0 = pltpu.async_copy(y_hbm.at[idx0_v], y0_v, sem)
        c1 = pltpu.async_copy(y_hbm.at[idx1_v], y1_v, sem)
        c2 = pltpu.async_copy(base_hbm.at[pl.ds(tok0, SB)], acc_v, sem)
        c3 = pltpu.async_copy(w0_hbm.at[pl.ds(tok0, SB)], w0_v, sem)
        c4 = pltpu.async_copy(w1_hbm.at[pl.ds(tok0, SB)], w1_v, sem)
        c0.wait()
        c1.wait()
        c2.wait()
        c3.wait()
        c4.wait()

        def row_body(r, _):
            w0s = w0_v[r]
            w1s = w1_v[r]
            for cch in range(DIM // 16):
                sl = pl.ds(cch * 16, 16)
                acc_v[r, sl] = (acc_v[r, sl] + w0s * y0_v[r, sl]
                                + w1s * y1_v[r, sl])
            return 0

        lax.fori_loop(0, SB, row_body, 0)
        pltpu.sync_copy(acc_v, out_hbm.at[pl.ds(tok0, SB)])


def _full(shape):
    return pl.BlockSpec(shape, lambda *_: tuple(0 for _ in shape))


def _row(dim):
    return pl.BlockSpec((1, dim), lambda *_: (0, 0))


def kernel(x, attn_norm_w, wq, w_kv_down, w_kv_up, wo, q_norm_w, k_norm_w,
           ffn_norm_w, gate_w, shared_w1, shared_w2, routed_w1, routed_w2):
    xf = x.reshape(T, DIM)
    c2, s2, pa, m = _rotary_consts()
    wku = w_kv_up.reshape(KV_LORA, 2, N_HEADS * HEAD_DIM)
    wkuk = wku[:, 0]
    wkuv = wku[:, 1]
    qw = jnp.tile(q_norm_w, N_HEADS).reshape(1, DIM)
    kw = jnp.tile(k_norm_w, N_HEADS).reshape(1, DIM)
    anw = attn_norm_w.reshape(1, DIM)
    fw = ffn_norm_w.reshape(1, DIM)

    tile = pl.BlockSpec((TB, DIM), lambda t: (t, 0))
    q, k, v = pl.pallas_call(
        _prologue_kern,
        grid=(T // TB,),
        in_specs=[tile, _row(DIM), _full((DIM, DIM)), _full((DIM, KV_LORA)),
                  _full((KV_LORA, DIM)), _full((KV_LORA, DIM)), tile, tile,
                  _full((DIM, DIM)), _full((DIM, N_HEADS)), _row(DIM),
                  _row(DIM)],
        out_specs=[pl.BlockSpec((N_HEADS, TB, HEAD_DIM),
                                lambda t: (0, t, 0))] * 3,
        out_shape=[jax.ShapeDtypeStruct((N_HEADS, T, HEAD_DIM),
                                        jnp.bfloat16)] * 3,
    )(xf, anw, wq, w_kv_down, wkuk, wkuv, c2, s2, pa, m, qw, kw)

    half = T // 2
    attn_lo = pl.pallas_call(
        functools.partial(_attn_kern, kw=half, qoff=0),
        grid=(N_HEADS, half // TA),
        in_specs=[pl.BlockSpec((1, TA, HEAD_DIM), lambda h, t: (h, t, 0)),
                  pl.BlockSpec((1, half, HEAD_DIM), lambda h, t: (h, 0, 0)),
                  pl.BlockSpec((1, half, HEAD_DIM), lambda h, t: (h, 0, 0))],
        out_specs=pl.BlockSpec((1, TA, HEAD_DIM), lambda h, t: (h, t, 0)),
        out_shape=jax.ShapeDtypeStruct((N_HEADS, half, HEAD_DIM),
                                       jnp.bfloat16),
    )(q, k, v)
    attn_hi = pl.pallas_call(
        functools.partial(_attn_kern, kw=T, qoff=half // TA),
        grid=(N_HEADS, half // TA),
        in_specs=[pl.BlockSpec((1, TA, HEAD_DIM),
                               lambda h, t: (h, t + half // TA, 0)),
                  pl.BlockSpec((1, T, HEAD_DIM), lambda h, t: (h, 0, 0)),
                  pl.BlockSpec((1, T, HEAD_DIM), lambda h, t: (h, 0, 0))],
        out_specs=pl.BlockSpec((1, TA, HEAD_DIM), lambda h, t: (h, t, 0)),
        out_shape=jax.ShapeDtypeStruct((N_HEADS, half, HEAD_DIM),
                                       jnp.bfloat16),
    )(q, k, v)
    attn = jnp.concatenate([attn_lo, attn_hi], axis=1)
    w1cat = jnp.concatenate([shared_w1[0], shared_w1[1]],
                            axis=1).astype(jnp.bfloat16)
    w2cat = jnp.concatenate([shared_w2[0], shared_w2[1]],
                            axis=0).astype(jnp.bfloat16)
    hn, base = pl.pallas_call(
        _oproj_shared_kern,
        grid=(T // TB,),
        in_specs=[pl.BlockSpec((N_HEADS, TB, HEAD_DIM),
                               lambda t: (0, t, 0)),
                  _full((DIM, DIM)), tile, _row(DIM),
                  _full((DIM, 2 * EXPERT_HIDDEN)),
                  _full((2 * EXPERT_HIDDEN, DIM))],
        out_specs=[tile, tile],
        out_shape=[jax.ShapeDtypeStruct((T, DIM), jnp.float32)] * 2,
    )(attn, wo.astype(jnp.bfloat16), xf, fw, w1cat, w2cat)

    aux, w0e, w1e, pos0, pos1, poff, cnt = pl.pallas_call(
        _gate_kern,
        grid=(1,),
        in_specs=[_full((T, DIM)), _full((DIM, NUM_EXPERTS))],
        out_specs=[_full((1, 1)), _full((T, 16)), _full((T, 16)),
                   _full((T, 1)), _full((T, 1)), _full((1, NUM_EXPERTS)),
                   _full((1, NUM_EXPERTS))],
        out_shape=[
            jax.ShapeDtypeStruct((1, 1), jnp.float32),
            jax.ShapeDtypeStruct((T, 16), jnp.float32),
            jax.ShapeDtypeStruct((T, 16), jnp.float32),
            jax.ShapeDtypeStruct((T, 1), jnp.int32),
            jax.ShapeDtypeStruct((T, 1), jnp.int32),
            jax.ShapeDtypeStruct((1, NUM_EXPERTS), jnp.int32),
            jax.ShapeDtypeStruct((1, NUM_EXPERTS), jnp.int32),
        ],
    )(hn, gate_w)

    pos0d = pos0.reshape(NW, TPW)
    pos1d = pos1.reshape(NW, TPW)
    g = _dispatch_sc(hn, pos0d, pos1d)

    y = pl.pallas_call(
        _expert_kern,
        grid=(NUM_EXPERTS,),
        in_specs=[pl.BlockSpec(memory_space=pltpu.SMEM),
                  pl.BlockSpec(memory_space=pltpu.SMEM),
                  _full((GR, DIM)),
                  pl.BlockSpec((1, DIM, EXPERT_HIDDEN), lambda e: (e, 0, 0)),
                  pl.BlockSpec((1, EXPERT_HIDDEN, DIM), lambda e: (e, 0, 0))],
        out_specs=_full((GR, DIM)),
        out_shape=jax.ShapeDtypeStruct((GR, DIM), jnp.float32),
    )(poff, cnt, g, routed_w1, routed_w2)

    pos0c = pos0.reshape(NW, TPW // SB, SB)
    pos1c = pos1.reshape(NW, TPW // SB, SB)
    out = _combine_sc(base, y, pos0c, pos1c, w0e, w1e)

    return out.reshape(x.shape), aux.reshape(())
